# Initial kernel scaffold; baseline (speedup 1.0000x reference)
import jax
import jax.numpy as jnp
from jax.experimental import pallas as pl


def kernel(inp, embed_table, pos_table):
    raise NotImplementedError("write your pallas kernel here")


# speedup vs baseline: 2.6177x; 2.6177x over previous
import functools

import jax
import jax.numpy as jnp
from jax import lax
from jax.experimental import pallas as pl
from jax.experimental.pallas import tpu as pltpu
from jax.experimental.pallas import tpu_sc as plsc

VOCAB = 100000
D = 128
B = 1024
S = 200

NC = 2
NS = 16
NW = NC * NS
ROWS_W = B // NW
CHUNK = 40
CPR = S // CHUNK
NCHUNK = ROWS_W * CPR
NBUF = 4
NV = D // 16


def _worker_body(inp_hbm, emb_hbm, pos_hbm, out_hbm,
                 idx_v, pos_idx_v, pos_v, gbufs, obufs, gsems, ssems, psem):
    wid = lax.axis_index("s") * NC + lax.axis_index("c")
    row0 = wid * ROWS_W

    pltpu.sync_copy(
        inp_hbm.at[pl.ds(pl.multiple_of(wid * ROWS_W * S, ROWS_W * S),
                         ROWS_W * S)], idx_v)
    pltpu.sync_copy(inp_hbm.at[pl.ds(0, S)], pos_idx_v)

    pltpu.async_copy(pos_hbm.at[pos_idx_v.at[pl.ds(0, 128)]],
                     pos_v.at[pl.ds(0, 128)], psem).wait()
    pltpu.async_copy(pos_hbm.at[pos_idx_v.at[pl.ds(128, 72)]],
                     pos_v.at[pl.ds(128, 72)], psem).wait()

    def start_gather(j, b):
        idx_slice = idx_v.at[pl.ds(pl.multiple_of(j * CHUNK, CHUNK), CHUNK)]
        pltpu.async_copy(emb_hbm.at[idx_slice], gbufs[b], gsems[b])

    for b in range(NBUF):
        start_gather(b, b)

    def outer(i, _):
        jo = i * NBUF
        for b in range(NBUF):
            j = jo + b
            pltpu.make_async_copy(emb_hbm.at[pl.ds(0, CHUNK)], gbufs[b],
                                  gsems[b]).wait()

            @pl.when(j >= NBUF)
            def _():
                pltpu.make_async_copy(obufs[b], out_hbm.at[0, pl.ds(0, CHUNK)],
                                      ssems[b]).wait()

            r = j // CPR
            c = j - r * CPR
            s0 = c * CHUNK

            def add_body(s, carry, b=b, s0=s0):
                for db in range(NV):
                    dsl = pl.ds(db * 16, 16)
                    obufs[b][s, dsl] = gbufs[b][s, dsl] + pos_v[s0 + s, dsl]
                return carry

            lax.fori_loop(0, CHUNK, add_body, 0, unroll=4)

            pltpu.async_copy(
                obufs[b],
                out_hbm.at[row0 + r,
                           pl.ds(pl.multiple_of(s0, CHUNK), CHUNK)],
                ssems[b])

            @pl.when(j + NBUF < NCHUNK)
            def _():
                start_gather(j + NBUF, b)
        return 0

    lax.fori_loop(0, NCHUNK // NBUF, outer, 0)

    for b in range(NBUF):
        pltpu.make_async_copy(obufs[b], out_hbm.at[0, pl.ds(0, CHUNK)],
                              ssems[b]).wait()


@functools.partial(jax.jit, static_argnums=())
def kernel(inp, embed_table, pos_table):
    mesh = plsc.VectorSubcoreMesh(core_axis_name="c", subcore_axis_name="s")
    scratch = (
        [pltpu.VMEM((ROWS_W * S,), jnp.int32),
         pltpu.VMEM((S,), jnp.int32),
         pltpu.VMEM((S, D), jnp.float32)]
        + [[pltpu.VMEM((CHUNK, D), jnp.float32) for _ in range(NBUF)]]
        + [[pltpu.VMEM((CHUNK, D), jnp.float32) for _ in range(NBUF)]]
        + [[pltpu.SemaphoreType.DMA for _ in range(NBUF)]]
        + [[pltpu.SemaphoreType.DMA for _ in range(NBUF)]]
        + [pltpu.SemaphoreType.DMA]
    )
    run = pl.kernel(
        _worker_body,
        out_type=jax.ShapeDtypeStruct((B, S, D), jnp.float32),
        mesh=mesh,
        scratch_types=scratch,
    )
    return run(inp.astype(jnp.int32).reshape(B * S), embed_table, pos_table)


# speedup vs baseline: 7.2092x; 2.7541x over previous
import functools

import jax
import jax.numpy as jnp
from jax import lax
from jax.experimental import pallas as pl
from jax.experimental.pallas import tpu as pltpu
from jax.experimental.pallas import tpu_sc as plsc

VOCAB = 100000
D = 128
B = 1024
S = 200

NC = 2
NS = 16
NW = NC * NS
ROWS_W = B // NW
CHUNK = 40
CPR = S // CHUNK
NCHUNK = ROWS_W * CPR
NBUF = 4
NV = D // 16


def _worker_body(inp_hbm, emb_hbm, pos_hbm, out_hbm,
                 idx_v, pos_idx_v, pos_v, gbufs, obufs, gsems, ssems, psem):
    wid = lax.axis_index("s") * NC + lax.axis_index("c")
    row0 = wid * ROWS_W

    pltpu.sync_copy(
        inp_hbm.at[pl.ds(pl.multiple_of(wid * ROWS_W * S, ROWS_W * S),
                         ROWS_W * S)], idx_v)
    pltpu.sync_copy(inp_hbm.at[pl.ds(0, S)], pos_idx_v)

    pltpu.async_copy(pos_hbm.at[pos_idx_v.at[pl.ds(0, 128)]],
                     pos_v.at[pl.ds(0, 128)], psem).wait()
    pltpu.async_copy(pos_hbm.at[pos_idx_v.at[pl.ds(128, 72)]],
                     pos_v.at[pl.ds(128, 72)], psem).wait()

    def start_gather(j, b):
        idx_slice = idx_v.at[pl.ds(pl.multiple_of(j * CHUNK, CHUNK), CHUNK)]
        pltpu.async_copy(emb_hbm.at[idx_slice], gbufs[b], gsems[b])

    for b in range(NBUF):
        start_gather(b, b)

    def outer(i, _):
        jo = i * NBUF
        for b in range(NBUF):
            j = jo + b
            pltpu.make_async_copy(emb_hbm.at[pl.ds(0, CHUNK)], gbufs[b],
                                  gsems[b]).wait()

            @pl.when(j >= NBUF)
            def _():
                pltpu.make_async_copy(obufs[b], out_hbm.at[0, pl.ds(0, CHUNK)],
                                      ssems[b]).wait()

            r = j // CPR
            c = j - r * CPR
            s0 = c * CHUNK

            @plsc.parallel_loop(0, CHUNK, unroll=4)
            def _(s, b=b, s0=s0):
                for db in range(NV):
                    dsl = pl.ds(db * 16, 16)
                    obufs[b][s, dsl] = gbufs[b][s, dsl] + pos_v[s0 + s, dsl]

            pltpu.async_copy(
                obufs[b],
                out_hbm.at[row0 + r,
                           pl.ds(pl.multiple_of(s0, CHUNK), CHUNK)],
                ssems[b])

            @pl.when(j + NBUF < NCHUNK)
            def _():
                start_gather(j + NBUF, b)
        return 0

    lax.fori_loop(0, NCHUNK // NBUF, outer, 0)

    for b in range(NBUF):
        pltpu.make_async_copy(obufs[b], out_hbm.at[0, pl.ds(0, CHUNK)],
                              ssems[b]).wait()


@functools.partial(jax.jit, static_argnums=())
def kernel(inp, embed_table, pos_table):
    mesh = plsc.VectorSubcoreMesh(core_axis_name="c", subcore_axis_name="s")
    scratch = (
        [pltpu.VMEM((ROWS_W * S,), jnp.int32),
         pltpu.VMEM((S,), jnp.int32),
         pltpu.VMEM((S, D), jnp.float32)]
        + [[pltpu.VMEM((CHUNK, D), jnp.float32) for _ in range(NBUF)]]
        + [[pltpu.VMEM((CHUNK, D), jnp.float32) for _ in range(NBUF)]]
        + [[pltpu.SemaphoreType.DMA for _ in range(NBUF)]]
        + [[pltpu.SemaphoreType.DMA for _ in range(NBUF)]]
        + [pltpu.SemaphoreType.DMA]
    )
    run = pl.kernel(
        _worker_body,
        out_type=jax.ShapeDtypeStruct((B, S, D), jnp.float32),
        mesh=mesh,
        scratch_types=scratch,
    )
    return run(inp.astype(jnp.int32).reshape(B * S), embed_table, pos_table)
